# baseline (speedup 1.0000x reference)
import jax
import jax.numpy as jnp
from jax import lax
from jax.experimental import pallas as pl
from jax.experimental.pallas import tpu as pltpu
from jax.experimental.pallas import tpu_sc as plsc

K = 64
I = 100000
B = 50
CW = 3200
TAILW = 896
LAM2 = 2.0 * 1e-4


def _body(y_hbm, ids_hbm, like_hbm, x_hbm, out_hbm,
          zbuf, ids_v, like_v, x_v, yblk, oblk, psum, gbuf,
          sem_in, sem_zero, sem_g, sem_o, sem_s):
    wid = lax.axis_index("c") * 16 + lax.axis_index("s")
    olo = wid * CW
    ohi = jnp.minimum(olo + CW, I)

    h_ids = pltpu.async_copy(ids_hbm, ids_v.at[pl.ds(0, B)], sem_in)
    h_like = pltpu.async_copy(like_hbm, like_v.at[pl.ds(0, B)], sem_in)
    h_x = pltpu.async_copy(x_hbm, x_v, sem_in)

    zeros16 = jnp.zeros((16,), jnp.float32)
    zidx = jnp.zeros((16,), jnp.int32)
    iota = lax.broadcasted_iota(jnp.int32, (16,), 0)

    def memset(i, c):
        for r in range(8):
            zbuf[r, pl.ds(i * 16, 16)] = zeros16
        return c

    lax.fori_loop(0, CW // 16, memset, 0)

    def zero_fire(base, width):
        for r in range(8):
            pltpu.async_copy(zbuf.at[pl.ds(0, 8), pl.ds(0, width)],
                             out_hbm.at[pl.ds(r * 8, 8), pl.ds(base, width)],
                             sem_zero)

    def zero_drain(base, width):
        for r in range(8):
            pltpu.make_async_copy(
                zbuf.at[pl.ds(0, 8), pl.ds(0, width)],
                out_hbm.at[pl.ds(r * 8, 8), pl.ds(base, width)],
                sem_zero).wait()

    @pl.when(wid < 31)
    def _():
        zero_fire(pl.multiple_of(wid * CW, 128), CW)

    @pl.when(wid == 31)
    def _():
        zero_fire(pl.multiple_of(wid * CW, 128), TAILW)

    h_ids.wait()
    h_like.wait()
    h_x.wait()

    xcs = [x_v[pl.ds(16 * ch, 16)] for ch in range(4)]

    def per_item_compute(b, c):
        mid = ids_v[pl.ds(b, 16)][0]

        @pl.when(jnp.logical_and(mid >= olo, mid < ohi))
        def _():
            col0 = pl.multiple_of((mid // 128) * 128, 128)
            cols = (mid - col0) + zidx
            pltpu.async_copy(y_hbm.at[pl.ds(0, K), pl.ds(col0, 128)],
                             yblk, sem_g).wait()
            ycs = [plsc.load_gather(yblk, [iota + 16 * ch, cols])
                   for ch in range(4)]
            psum[pl.ds(0, 16)] = zeros16
            for ch in range(4):
                plsc.addupdate_scatter(psum, [zidx], xcs[ch] * ycs[ch])
            pred = psum[pl.ds(0, 16)][0]
            coeff = (pred - like_v[pl.ds(b, 16)][0]) * jnp.float32(2.0 / B)
            for ch in range(4):
                gbuf[pl.ds(b * 64 + 16 * ch, 16)] = (
                    coeff * xcs[ch] + jnp.float32(LAM2 / B) * ycs[ch])

        return c

    @pl.when(wid < 31)
    def _():
        zero_drain(pl.multiple_of(wid * CW, 128), CW)

    @pl.when(wid == 31)
    def _():
        zero_drain(pl.multiple_of(wid * CW, 128), TAILW)

    lax.fori_loop(0, B, per_item_compute, 0)

    def per_item_scatter(b, c):
        mid = ids_v[pl.ds(b, 16)][0]

        @pl.when(jnp.logical_and(mid >= olo, mid < ohi))
        def _():
            col0 = pl.multiple_of((mid // 128) * 128, 128)
            cols = (mid - col0) + zidx
            pltpu.async_copy(out_hbm.at[pl.ds(0, K), pl.ds(col0, 128)],
                             oblk, sem_o).wait()
            for ch in range(4):
                plsc.store_scatter(oblk, [iota + 16 * ch, cols],
                                   gbuf[pl.ds(b * 64 + 16 * ch, 16)])
            pltpu.async_copy(oblk,
                             out_hbm.at[pl.ds(0, K), pl.ds(col0, 128)],
                             sem_s).wait()

        return c

    lax.fori_loop(0, B, per_item_scatter, 0)


_call = pl.kernel(
    _body,
    out_type=jax.ShapeDtypeStruct((K, I), jnp.float32),
    mesh=plsc.VectorSubcoreMesh(core_axis_name="c", subcore_axis_name="s"),
    compiler_params=pltpu.CompilerParams(needs_layout_passes=False,
                                         use_tc_tiling_on_sc=True),
    scratch_types=[
        pltpu.VMEM((8, CW), jnp.float32),
        pltpu.VMEM((80,), jnp.int32),
        pltpu.VMEM((80,), jnp.float32),
        pltpu.VMEM((64,), jnp.float32),
        pltpu.VMEM((K, 128), jnp.float32),
        pltpu.VMEM((K, 128), jnp.float32),
        pltpu.VMEM((16,), jnp.float32),
        pltpu.VMEM((B * 64,), jnp.float32),
        pltpu.SemaphoreType.DMA,
        pltpu.SemaphoreType.DMA,
        pltpu.SemaphoreType.DMA,
        pltpu.SemaphoreType.DMA,
        pltpu.SemaphoreType.DMA,
    ],
)


def kernel(global_Y, movie_ids, is_like, X):
    return _call(global_Y, movie_ids, is_like, X.reshape(-1))


# speedup vs baseline: 1.1460x; 1.1460x over previous
import jax
import jax.numpy as jnp
from jax import lax
from jax.experimental import pallas as pl
from jax.experimental.pallas import tpu as pltpu
from jax.experimental.pallas import tpu_sc as plsc

K = 64
I = 100000
B = 50
CW = 3200
TAILW = 896
LAM2 = 2.0 * 1e-4


def _body(y_hbm, ids_hbm, like_hbm, x_hbm, out_hbm,
          zbuf, ids_v, like_v, x_v, yblk, oblk, psum,
          sem_in, sem_zero, sem_g, sem_o, sem_s):
    wid = lax.axis_index("c") * 16 + lax.axis_index("s")
    olo = wid * CW
    ohi = jnp.minimum(olo + CW, I)

    h_ids = pltpu.async_copy(ids_hbm, ids_v.at[pl.ds(0, B)], sem_in)
    h_like = pltpu.async_copy(like_hbm, like_v.at[pl.ds(0, B)], sem_in)
    h_x = pltpu.async_copy(x_hbm, x_v, sem_in)

    zeros16 = jnp.zeros((16,), jnp.float32)
    zidx = jnp.zeros((16,), jnp.int32)
    iota = lax.broadcasted_iota(jnp.int32, (16,), 0)

    def memset(i, c):
        for r in range(8):
            zbuf[r, pl.ds(i * 16, 16)] = zeros16
        return c

    lax.fori_loop(0, CW // 16, memset, 0)

    def zero_fire(base, width):
        for r in range(8):
            pltpu.async_copy(zbuf.at[pl.ds(0, 8), pl.ds(0, width)],
                             out_hbm.at[pl.ds(r * 8, 8), pl.ds(base, width)],
                             sem_zero)

    def zero_drain(base, width):
        for r in range(8):
            pltpu.make_async_copy(
                zbuf.at[pl.ds(0, 8), pl.ds(0, width)],
                out_hbm.at[pl.ds(r * 8, 8), pl.ds(base, width)],
                sem_zero).wait()

    @pl.when(wid < 31)
    def _():
        zero_fire(pl.multiple_of(wid * CW, 128), CW)

    @pl.when(wid == 31)
    def _():
        zero_fire(pl.multiple_of(wid * CW, 128), TAILW)

    h_ids.wait()
    h_like.wait()
    h_x.wait()

    xcs = [x_v[pl.ds(16 * ch, 16)] for ch in range(4)]

    @pl.when(wid < 31)
    def _():
        zero_drain(pl.multiple_of(wid * CW, 128), CW)

    @pl.when(wid == 31)
    def _():
        zero_drain(pl.multiple_of(wid * CW, 128), TAILW)

    def per_item(b, c):
        mid = ids_v[pl.ds(b, 16)][0]

        @pl.when(jnp.logical_and(mid >= olo, mid < ohi))
        def _():
            col0 = pl.multiple_of((mid // 128) * 128, 128)
            cols = (mid - col0) + zidx
            pltpu.async_copy(y_hbm.at[pl.ds(0, K), pl.ds(col0, 128)],
                             yblk, sem_g)
            pltpu.async_copy(out_hbm.at[pl.ds(0, K), pl.ds(col0, 128)],
                             oblk, sem_o)
            pltpu.make_async_copy(y_hbm.at[pl.ds(0, K), pl.ds(col0, 128)],
                                  yblk, sem_g).wait()
            ycs = [plsc.load_gather(yblk, [iota + 16 * ch, cols])
                   for ch in range(4)]
            psum[pl.ds(0, 16)] = zeros16
            for ch in range(4):
                plsc.addupdate_scatter(psum, [zidx], xcs[ch] * ycs[ch])
            pred = psum[pl.ds(0, 16)][0]
            coeff = (pred - like_v[pl.ds(b, 16)][0]) * jnp.float32(2.0 / B)
            pltpu.make_async_copy(out_hbm.at[pl.ds(0, K), pl.ds(col0, 128)],
                                  oblk, sem_o).wait()
            for ch in range(4):
                gc = coeff * xcs[ch] + jnp.float32(LAM2 / B) * ycs[ch]
                plsc.store_scatter(oblk, [iota + 16 * ch, cols], gc)
            pltpu.async_copy(oblk,
                             out_hbm.at[pl.ds(0, K), pl.ds(col0, 128)],
                             sem_s).wait()

        return c

    lax.fori_loop(0, B, per_item, 0)


_call = pl.kernel(
    _body,
    out_type=jax.ShapeDtypeStruct((K, I), jnp.float32),
    mesh=plsc.VectorSubcoreMesh(core_axis_name="c", subcore_axis_name="s"),
    compiler_params=pltpu.CompilerParams(needs_layout_passes=False,
                                         use_tc_tiling_on_sc=True),
    scratch_types=[
        pltpu.VMEM((8, CW), jnp.float32),
        pltpu.VMEM((80,), jnp.int32),
        pltpu.VMEM((80,), jnp.float32),
        pltpu.VMEM((64,), jnp.float32),
        pltpu.VMEM((K, 128), jnp.float32),
        pltpu.VMEM((K, 128), jnp.float32),
        pltpu.VMEM((16,), jnp.float32),
        pltpu.SemaphoreType.DMA,
        pltpu.SemaphoreType.DMA,
        pltpu.SemaphoreType.DMA,
        pltpu.SemaphoreType.DMA,
        pltpu.SemaphoreType.DMA,
    ],
)


def kernel(global_Y, movie_ids, is_like, X):
    return _call(global_Y, movie_ids, is_like, X.reshape(-1))


# speedup vs baseline: 1.1470x; 1.0009x over previous
import jax
import jax.numpy as jnp
from jax import lax
from jax.experimental import pallas as pl
from jax.experimental.pallas import tpu as pltpu
from jax.experimental.pallas import tpu_sc as plsc

K = 64
I = 100000
B = 50
CW = 3200
TAILW = 896
LAM2 = 2.0 * 1e-4


def _body(y_hbm, ids_hbm, like_hbm, x_hbm, out_hbm,
          zbuf, ids_v, like_v, x_v, yblk, oblk, psum,
          sem_in, sem_zero, sem_g, sem_s):
    wid = lax.axis_index("c") * 16 + lax.axis_index("s")
    olo = wid * CW
    ohi = jnp.minimum(olo + CW, I)

    h_ids = pltpu.async_copy(ids_hbm, ids_v.at[pl.ds(0, B)], sem_in)
    h_like = pltpu.async_copy(like_hbm, like_v.at[pl.ds(0, B)], sem_in)
    h_x = pltpu.async_copy(x_hbm, x_v, sem_in)

    zeros16 = jnp.zeros((16,), jnp.float32)
    zidx = jnp.zeros((16,), jnp.int32)
    iota = lax.broadcasted_iota(jnp.int32, (16,), 0)

    def memset(i, c):
        for r in range(8):
            zbuf[r, pl.ds(i * 16, 16)] = zeros16
        return c

    lax.fori_loop(0, CW // 16, memset, 0)

    def zero_fire(base, width):
        for r in range(8):
            pltpu.async_copy(zbuf.at[pl.ds(0, 8), pl.ds(0, width)],
                             out_hbm.at[pl.ds(r * 8, 8), pl.ds(base, width)],
                             sem_zero)

    def zero_drain(base, width):
        for r in range(8):
            pltpu.make_async_copy(
                zbuf.at[pl.ds(0, 8), pl.ds(0, width)],
                out_hbm.at[pl.ds(r * 8, 8), pl.ds(base, width)],
                sem_zero).wait()

    h_ids.wait()
    h_like.wait()
    h_x.wait()

    xcs = [x_v[pl.ds(16 * ch, 16)] for ch in range(4)]

    @pl.when(wid < 31)
    def _():
        zero_fire(pl.multiple_of(wid * CW, 128), CW)

    @pl.when(wid == 31)
    def _():
        zero_fire(pl.multiple_of(wid * CW, 128), TAILW)

    @pl.when(wid < 31)
    def _():
        zero_drain(pl.multiple_of(wid * CW, 128), CW)

    @pl.when(wid == 31)
    def _():
        zero_drain(pl.multiple_of(wid * CW, 128), TAILW)

    def per_item(b, c):
        mid = ids_v[pl.ds(b, 16)][0]

        @pl.when(jnp.logical_and(mid >= olo, mid < ohi))
        def _():
            col0 = pl.multiple_of((mid // 128) * 128, 128)
            cols = (mid - col0) + zidx
            pltpu.async_copy(y_hbm.at[pl.ds(0, K), pl.ds(col0, 128)],
                             yblk, sem_g)
            pltpu.async_copy(out_hbm.at[pl.ds(0, K), pl.ds(col0, 128)],
                             oblk, sem_g)
            pltpu.make_async_copy(y_hbm.at[pl.ds(0, K), pl.ds(col0, 128)],
                                  yblk, sem_g).wait()
            pltpu.make_async_copy(out_hbm.at[pl.ds(0, K), pl.ds(col0, 128)],
                                  oblk, sem_g).wait()
            ycs = [plsc.load_gather(yblk, [iota + 16 * ch, cols])
                   for ch in range(4)]
            psum[pl.ds(0, 16)] = zeros16
            for ch in range(4):
                plsc.addupdate_scatter(psum, [zidx], xcs[ch] * ycs[ch])
            pred = psum[pl.ds(0, 16)][0]
            coeff = (pred - like_v[pl.ds(b, 16)][0]) * jnp.float32(2.0 / B)
            for ch in range(4):
                gc = coeff * xcs[ch] + jnp.float32(LAM2 / B) * ycs[ch]
                plsc.store_scatter(oblk, [iota + 16 * ch, cols], gc)
            pltpu.async_copy(oblk,
                             out_hbm.at[pl.ds(0, K), pl.ds(col0, 128)],
                             sem_s).wait()

        return c

    lax.fori_loop(0, B, per_item, 0)


_call = pl.kernel(
    _body,
    out_type=jax.ShapeDtypeStruct((K, I), jnp.float32),
    mesh=plsc.VectorSubcoreMesh(core_axis_name="c", subcore_axis_name="s"),
    compiler_params=pltpu.CompilerParams(needs_layout_passes=False,
                                         use_tc_tiling_on_sc=True),
    scratch_types=[
        pltpu.VMEM((8, CW), jnp.float32),
        pltpu.VMEM((80,), jnp.int32),
        pltpu.VMEM((80,), jnp.float32),
        pltpu.VMEM((64,), jnp.float32),
        pltpu.VMEM((K, 128), jnp.float32),
        pltpu.VMEM((K, 128), jnp.float32),
        pltpu.VMEM((16,), jnp.float32),
        pltpu.SemaphoreType.DMA,
        pltpu.SemaphoreType.DMA,
        pltpu.SemaphoreType.DMA,
        pltpu.SemaphoreType.DMA,
    ],
)


def kernel(global_Y, movie_ids, is_like, X):
    return _call(global_Y, movie_ids, is_like, X.reshape(-1))
